# baseline (speedup 1.0000x reference)
import functools

import jax
import jax.numpy as jnp
from jax import lax
from jax.experimental import pallas as pl
from jax.experimental.pallas import tpu as pltpu
from jax.experimental.pallas import tpu_sc as plsc

E = 8
TOP_K = 2
D = 2048
F = 1408
T = 2048
TM = 128
NT = (T * TOP_K) // TM + (E - 1)
ROWS = NT * TM


def _router_body(x_ref, wg_ref, eids_ref, w01_ref):
    l = jnp.dot(x_ref[...], wg_ref[...], preferred_element_type=jnp.float32)
    lane = jax.lax.broadcasted_iota(jnp.int32, l.shape, 1)
    l = jnp.where(lane < E, l, -1e30)
    m1 = jnp.max(l, axis=1, keepdims=True)
    a1 = jnp.min(jnp.where(l == m1, lane, E), axis=1, keepdims=True)
    l2 = jnp.where(lane == a1, -1e30, l)
    m2 = jnp.max(l2, axis=1, keepdims=True)
    a2 = jnp.min(jnp.where(l2 == m2, lane, E), axis=1, keepdims=True)
    w0 = 1.0 / (1.0 + jnp.exp(m2 - m1))
    eids_ref[0] = a1
    eids_ref[1] = a2
    w01_ref[0] = w0
    w01_ref[1] = 1.0 - w0


def _router(x, wg_pad):
    eids, w01 = pl.pallas_call(
        _router_body,
        grid=(T // TM,),
        in_specs=[
            pl.BlockSpec((TM, D), lambda i: (i, 0)),
            pl.BlockSpec((D, 128), lambda i: (0, 0)),
        ],
        out_specs=[
            pl.BlockSpec((2, TM, 1), lambda i: (0, i, 0)),
            pl.BlockSpec((2, TM, 1), lambda i: (0, i, 0)),
        ],
        out_shape=[
            jax.ShapeDtypeStruct((2, T, 1), jnp.int32),
            jax.ShapeDtypeStruct((2, T, 1), jnp.float32),
        ],
    )(x, wg_pad)
    return eids.reshape(2 * T), w01.reshape(2 * T)


def _gateup_body(emap_ref, tmap_ref, af_ref, xs_ref, wg_ref, wu_ref, h_ref):
    i = pl.program_id(0)

    @pl.when(af_ref[i] == 1)
    def _():
        x = xs_ref[...]
        g = jnp.dot(x, wg_ref[0], preferred_element_type=jnp.float32)
        u = jnp.dot(x, wu_ref[0], preferred_element_type=jnp.float32)
        h_ref[...] = (g * jax.nn.sigmoid(g) * u).astype(jnp.bfloat16)


def _down_body(emap_ref, tmap_ref, af_ref, h_ref, wd_ref, out_ref):
    i = pl.program_id(0)

    @pl.when(af_ref[i] == 1)
    def _():
        h = h_ref[...].astype(jnp.float32)
        out_ref[...] = jnp.dot(h, wd_ref[0], preferred_element_type=jnp.float32)


def _grouped_ffn(xs, w_gate, w_up, w_down, emap, tmap, af):
    h = pl.pallas_call(
        _gateup_body,
        grid_spec=pltpu.PrefetchScalarGridSpec(
            num_scalar_prefetch=3,
            grid=(NT,),
            in_specs=[
                pl.BlockSpec((TM, D), lambda i, em, tm, af_: (tm[i], 0)),
                pl.BlockSpec((1, D, F), lambda i, em, tm, af_: (em[i], 0, 0)),
                pl.BlockSpec((1, D, F), lambda i, em, tm, af_: (em[i], 0, 0)),
            ],
            out_specs=pl.BlockSpec((TM, F), lambda i, em, tm, af_: (tm[i], 0)),
        ),
        out_shape=jax.ShapeDtypeStruct((ROWS, F), jnp.bfloat16),
        compiler_params=pltpu.CompilerParams(
            vmem_limit_bytes=62 * 1024 * 1024,
        ),
    )(emap, tmap, af, xs, w_gate, w_up)
    return pl.pallas_call(
        _down_body,
        grid_spec=pltpu.PrefetchScalarGridSpec(
            num_scalar_prefetch=3,
            grid=(NT,),
            in_specs=[
                pl.BlockSpec((TM, F), lambda i, em, tm, af_: (tm[i], 0)),
                pl.BlockSpec((1, F, D), lambda i, em, tm, af_: (em[i], 0, 0)),
            ],
            out_specs=pl.BlockSpec((TM, D), lambda i, em, tm, af_: (tm[i], 0)),
        ),
        out_shape=jax.ShapeDtypeStruct((ROWS, D), jnp.float32),
        compiler_params=pltpu.CompilerParams(
            vmem_limit_bytes=62 * 1024 * 1024,
        ),
    )(emap, tmap, af, h, w_down)


NW = 32
NTP = 48


def _sc_plan_gather_body(eids_hbm, x_hbm, xs_hbm, pos_hbm, emap_hbm, tmap_hbm,
                         af_hbm, eid_v, posA_v, posB_v, plan_v,
                         r0, r1, r2, sr0, sr1, sr2, ss0, ss1, ss2):
    nc = 2
    wid = lax.axis_index("s") * nc + lax.axis_index("c")
    lanes = lax.iota(jnp.int32, 16)
    pltpu.sync_copy(eids_hbm, eid_v)
    zero = jnp.zeros((16,), jnp.int32)
    HC = T // NW

    def _count_w(w, carry):
        cnt_all, baseA, baseB = carry
        snapA = jnp.where(w == wid, cnt_all, zero)
        snapB = jnp.where(w == NW + wid, cnt_all, zero)
        chunk_cnt = zero
        for k in range(HC // 16):
            v = eid_v[pl.ds(w * HC + k * 16, 16)]
            for e in range(E):
                pc = jnp.sum((v == e).astype(jnp.int32))
                chunk_cnt = chunk_cnt + jnp.where(lanes == e, pc, 0)
        return cnt_all + chunk_cnt, baseA + snapA, baseB + snapB

    cnt_all, baseA, baseB = lax.fori_loop(0, 2 * NW, _count_w, (zero, zero, zero))
    ntiles = (cnt_all + TM - 1) // TM
    inc = plsc.cumsum(ntiles)
    tstart = (inc - ntiles) * TM
    tok0 = wid * HC
    for slot, mybase, pos_v in ((0, tstart + baseA, posA_v), (1, tstart + baseB, posB_v)):
        run = zero
        for k in range(HC // 16):
            v = eid_v[pl.ds(slot * T + tok0 + k * 16, 16)]
            pos_k = zero
            for e in range(E):
                m = v == e
                mi = m.astype(jnp.int32)
                pref = plsc.cumsum(mi) - mi
                base_sc = jnp.sum(jnp.where(lanes == e, mybase + run, 0))
                pos_k = jnp.where(m, base_sc + pref, pos_k)
                run = run + jnp.where(lanes == e, jnp.sum(mi), 0)
            pos_v[pl.ds(k * 16, 16)] = pos_k
        pltpu.sync_copy(pos_v, pos_hbm.at[pl.ds(slot * T + tok0, HC)])
    GR = 16
    NG = HC // GR
    rows = (r0, r1, r2)
    sr = (sr0, sr1, sr2)
    ss = (ss0, ss1, ss2)
    reads = {}
    for j in range(3):
        reads[j] = pltpu.async_copy(x_hbm.at[pl.ds(tok0 + j * GR, GR)], rows[j], sr[j])
    pend = {}
    for k in range(NG):
        b = k % 3
        reads.pop(k).wait()
        ia = posA_v[pl.ds(k * GR, GR)]
        ib = posB_v[pl.ds(k * GR, GR)]
        pend[k] = (pltpu.async_copy(rows[b], xs_hbm.at[ia], ss[b]),
                   pltpu.async_copy(rows[b], xs_hbm.at[ib], ss[b]))
        if k + 3 < NG:
            for hnd in pend.pop(k):
                hnd.wait()
            reads[k + 3] = pltpu.async_copy(
                x_hbm.at[pl.ds(tok0 + (k + 3) * GR, GR)], rows[b], sr[b])
    for pair in pend.values():
        for hnd in pair:
            hnd.wait()
    @pl.when(wid == 0)
    def _():
        nact = jnp.sum(jnp.where(lanes == E - 1, inc, 0))
        e_last = zero
        for e in range(E):
            te = jnp.sum(jnp.where(lanes == e, inc, 0))
            e_last = e_last + jnp.where(nact - 1 >= te, 1, 0)
        for half in range(NTP // 16):
            j = lax.iota(jnp.int32, 16) + half * 16
            ej = zero
            for e in range(E):
                te = jnp.sum(jnp.where(lanes == e, inc, 0))
                ej = ej + jnp.where(j >= te, 1, 0)
            act = j < nact
            plan_v[pl.ds(0, 16)] = jnp.where(act, ej, e_last)
            plan_v[pl.ds(16, 16)] = jnp.minimum(j, nact - 1)
            plan_v[pl.ds(32, 16)] = act.astype(jnp.int32)
            pltpu.sync_copy(plan_v.at[pl.ds(0, 16)], emap_hbm.at[pl.ds(half * 16, 16)])
            pltpu.sync_copy(plan_v.at[pl.ds(16, 16)], tmap_hbm.at[pl.ds(half * 16, 16)])
            pltpu.sync_copy(plan_v.at[pl.ds(32, 16)], af_hbm.at[pl.ds(half * 16, 16)])


def _sc_plan_gather(eids, x):
    mesh = plsc.VectorSubcoreMesh(core_axis_name="c", subcore_axis_name="s")
    f = pl.kernel(
        _sc_plan_gather_body,
        mesh=mesh,
        out_type=[
            jax.ShapeDtypeStruct((ROWS, D), jnp.float32),
            jax.ShapeDtypeStruct((2 * T,), jnp.int32),
            jax.ShapeDtypeStruct((NTP,), jnp.int32),
            jax.ShapeDtypeStruct((NTP,), jnp.int32),
            jax.ShapeDtypeStruct((NTP,), jnp.int32),
        ],
        scratch_types=[
            pltpu.VMEM((2 * T,), jnp.int32),
            pltpu.VMEM((T // NW,), jnp.int32),
            pltpu.VMEM((T // NW,), jnp.int32),
            pltpu.VMEM((48,), jnp.int32),
            pltpu.VMEM((16, D), jnp.float32),
            pltpu.VMEM((16, D), jnp.float32),
            pltpu.VMEM((16, D), jnp.float32),
        ] + [pltpu.SemaphoreType.DMA] * 6,
        compiler_params=pltpu.CompilerParams(needs_layout_passes=False),
    )
    return f(eids, x)


def _sc_combine_body(x_hbm, ys_hbm, pos_hbm, w_hbm, y_hbm,
                     posA_v, posB_v, wAB_v,
                     rx0, rx1, ra0, ra1, rb0, rb1,
                     sx0, sx1, sa0, sa1, sb0, sb1, sw0, sw1):
    nc = 2
    wid = lax.axis_index("s") * nc + lax.axis_index("c")
    ntok = T // NW
    GR = 8
    NG = ntok // GR
    tok0 = wid * ntok
    rx = (rx0, rx1)
    ra = (ra0, ra1)
    rb = (rb0, rb1)
    sx = (sx0, sx1)
    sa = (sa0, sa1)
    sb = (sb0, sb1)
    sw = (sw0, sw1)
    pltpu.sync_copy(pos_hbm.at[pl.ds(tok0, ntok)], posA_v)
    pltpu.sync_copy(pos_hbm.at[pl.ds(T + tok0, ntok)], posB_v)
    pltpu.sync_copy(w_hbm.at[pl.ds(tok0, ntok)], wAB_v.at[pl.ds(0, ntok)])
    pltpu.sync_copy(w_hbm.at[pl.ds(T + tok0, ntok)], wAB_v.at[pl.ds(ntok, ntok)])

    def start_inputs(g, b):
        return (
            pltpu.async_copy(x_hbm.at[pl.ds(tok0 + g * GR, GR)], rx[b], sx[b]),
            pltpu.async_copy(ys_hbm.at[posA_v.at[pl.ds(g * GR, GR)]], ra[b], sa[b]),
            pltpu.async_copy(ys_hbm.at[posB_v.at[pl.ds(g * GR, GR)]], rb[b], sb[b]),
        )

    pend_in = {0: start_inputs(0, 0)}
    pend_w = {}
    for g in range(NG):
        b = g % 2
        for hnd in pend_in.pop(g):
            hnd.wait()
        if g + 1 < NG:
            bn = 1 - b
            if bn in pend_w:
                pend_w.pop(bn).wait()
            pend_in[g + 1] = start_inputs(g + 1, bn)
        wa16 = wAB_v[pl.ds((g // 2) * 16, 16)]
        wb16 = wAB_v[pl.ds(ntok + (g // 2) * 16, 16)]
        for r in range(GR):
            lane = r + GR * (g % 2)
            ridx = jnp.full((16,), lane, jnp.int32)
            was = lax.gather(wa16, ridx[:, None],
                             lax.GatherDimensionNumbers((), (0,), (0,)), (1,),
                             mode=lax.GatherScatterMode.PROMISE_IN_BOUNDS)
            wbs = lax.gather(wb16, ridx[:, None],
                             lax.GatherDimensionNumbers((), (0,), (0,)), (1,),
                             mode=lax.GatherScatterMode.PROMISE_IN_BOUNDS)

            def _col(ci, _):
                for u in range(8):
                    sl = pl.ds(ci * 128 + u * 16, 16)
                    rx[b][r, sl] = rx[b][r, sl] + was * ra[b][r, sl] + wbs * rb[b][r, sl]
                return 0

            lax.fori_loop(0, D // 128, _col, 0)
        pend_w[b] = pltpu.async_copy(rx[b], y_hbm.at[pl.ds(tok0 + g * GR, GR)], sw[b])
    for hnd in pend_w.values():
        hnd.wait()


def _sc_combine(x, ys, pos, w01):
    mesh = plsc.VectorSubcoreMesh(core_axis_name="c", subcore_axis_name="s")
    ntok = T // NW
    row = pltpu.VMEM((8, D), jnp.float32)
    f = pl.kernel(
        _sc_combine_body,
        mesh=mesh,
        out_type=jax.ShapeDtypeStruct((T, D), jnp.float32),
        scratch_types=[
            pltpu.VMEM((ntok,), jnp.int32),
            pltpu.VMEM((ntok,), jnp.int32),
            pltpu.VMEM((2 * ntok,), jnp.float32),
            row, row, row, row, row, row,
        ] + [pltpu.SemaphoreType.DMA] * 8,
        compiler_params=pltpu.CompilerParams(needs_layout_passes=False),
    )
    return f(x, ys, pos, w01)


def _plan(idx0, idx1):
    eids = jnp.concatenate([idx0, idx1])
    tokens = jnp.concatenate([jnp.arange(T, dtype=jnp.int32)] * 2)
    counts = jnp.bincount(eids, length=E)
    ntiles = (counts + TM - 1) // TM
    tile_cum = jnp.cumsum(ntiles)
    tstart = (tile_cum - ntiles) * TM
    nact = tile_cum[-1]
    perm = jnp.argsort(eids, stable=True)
    cnt_excl = jnp.cumsum(counts) - counts
    se = eids[perm]
    rank = jnp.arange(2 * T, dtype=jnp.int32) - cnt_excl[se]
    row_sorted = tstart[se].astype(jnp.int32) + rank
    pos = jnp.zeros((2 * T,), jnp.int32).at[perm].set(row_sorted)
    src = jnp.zeros((ROWS,), jnp.int32).at[row_sorted].set(tokens[perm])
    j = jnp.arange(NT, dtype=jnp.int32)
    ej = jnp.searchsorted(tile_cum, j, side="right").astype(jnp.int32)
    af = (j < nact).astype(jnp.int32)
    emap = jnp.where(af == 1, jnp.minimum(ej, E - 1), jnp.minimum(ej, E - 1))
    emap = jnp.where(af == 1, emap, emap[jnp.maximum(nact - 1, 0)])
    tmap = jnp.minimum(j, nact - 1).astype(jnp.int32)
    return pos[:T], pos[T:], src, emap, tmap, af


def kernel(hidden_states, Wg, W_gate, W_up, W_down):
    orig_shape = hidden_states.shape
    x = hidden_states.reshape(-1, orig_shape[-1])
    wg_pad = jnp.zeros((D, 128), jnp.float32).at[:, :E].set(Wg)
    eids, w01 = _router(x, wg_pad)
    xs, pos, emap, tmap, af = _sc_plan_gather(eids, x)
    ys = _grouped_ffn(xs, W_gate, W_up, W_down, emap, tmap, af)
    y = _sc_combine(x, ys, pos, w01)
    return y.reshape(orig_shape)


# speedup vs baseline: 1.0428x; 1.0428x over previous
import functools

import jax
import jax.numpy as jnp
from jax import lax
from jax.experimental import pallas as pl
from jax.experimental.pallas import tpu as pltpu
from jax.experimental.pallas import tpu_sc as plsc

E = 8
TOP_K = 2
D = 2048
F = 1408
T = 2048
TM = 256
NT = (T * TOP_K) // TM + (E - 1)
ROWS = NT * TM


def _router_body(x_ref, wg_ref, eids_ref, w01_ref):
    l = jnp.dot(x_ref[...], wg_ref[...], preferred_element_type=jnp.float32)
    lane = jax.lax.broadcasted_iota(jnp.int32, l.shape, 1)
    l = jnp.where(lane < E, l, -1e30)
    m1 = jnp.max(l, axis=1, keepdims=True)
    a1 = jnp.min(jnp.where(l == m1, lane, E), axis=1, keepdims=True)
    l2 = jnp.where(lane == a1, -1e30, l)
    m2 = jnp.max(l2, axis=1, keepdims=True)
    a2 = jnp.min(jnp.where(l2 == m2, lane, E), axis=1, keepdims=True)
    w0 = 1.0 / (1.0 + jnp.exp(m2 - m1))
    eids_ref[0] = a1
    eids_ref[1] = a2
    w01_ref[0] = w0
    w01_ref[1] = 1.0 - w0


def _router(x, wg_pad):
    eids, w01 = pl.pallas_call(
        _router_body,
        grid=(T // TM,),
        in_specs=[
            pl.BlockSpec((TM, D), lambda i: (i, 0)),
            pl.BlockSpec((D, 128), lambda i: (0, 0)),
        ],
        out_specs=[
            pl.BlockSpec((2, TM, 1), lambda i: (0, i, 0)),
            pl.BlockSpec((2, TM, 1), lambda i: (0, i, 0)),
        ],
        out_shape=[
            jax.ShapeDtypeStruct((2, T, 1), jnp.int32),
            jax.ShapeDtypeStruct((2, T, 1), jnp.float32),
        ],
    )(x, wg_pad)
    return eids.reshape(2 * T), w01.reshape(2 * T)


def _gateup_body(emap_ref, tmap_ref, af_ref, xs_ref, wg_ref, wu_ref, h_ref):
    i = pl.program_id(0)

    @pl.when(af_ref[i] == 1)
    def _():
        x = xs_ref[...]
        g = jnp.dot(x, wg_ref[0], preferred_element_type=jnp.float32)
        u = jnp.dot(x, wu_ref[0], preferred_element_type=jnp.float32)
        h_ref[...] = (g * jax.nn.sigmoid(g) * u).astype(jnp.bfloat16)


def _down_body(emap_ref, tmap_ref, af_ref, h_ref, wd_ref, out_ref):
    i = pl.program_id(0)

    @pl.when(af_ref[i] == 1)
    def _():
        h = h_ref[...].astype(jnp.float32)
        out_ref[...] = jnp.dot(h, wd_ref[0], preferred_element_type=jnp.float32)


def _grouped_ffn(xs, w_gate, w_up, w_down, emap, tmap, af):
    h = pl.pallas_call(
        _gateup_body,
        grid_spec=pltpu.PrefetchScalarGridSpec(
            num_scalar_prefetch=3,
            grid=(NT,),
            in_specs=[
                pl.BlockSpec((TM, D), lambda i, em, tm, af_: (tm[i], 0)),
                pl.BlockSpec((1, D, F), lambda i, em, tm, af_: (em[i], 0, 0)),
                pl.BlockSpec((1, D, F), lambda i, em, tm, af_: (em[i], 0, 0)),
            ],
            out_specs=pl.BlockSpec((TM, F), lambda i, em, tm, af_: (tm[i], 0)),
        ),
        out_shape=jax.ShapeDtypeStruct((ROWS, F), jnp.bfloat16),
        compiler_params=pltpu.CompilerParams(
            vmem_limit_bytes=62 * 1024 * 1024,
        ),
    )(emap, tmap, af, xs, w_gate, w_up)
    return pl.pallas_call(
        _down_body,
        grid_spec=pltpu.PrefetchScalarGridSpec(
            num_scalar_prefetch=3,
            grid=(NT,),
            in_specs=[
                pl.BlockSpec((TM, F), lambda i, em, tm, af_: (tm[i], 0)),
                pl.BlockSpec((1, F, D), lambda i, em, tm, af_: (em[i], 0, 0)),
            ],
            out_specs=pl.BlockSpec((TM, D), lambda i, em, tm, af_: (tm[i], 0)),
        ),
        out_shape=jax.ShapeDtypeStruct((ROWS, D), jnp.float32),
        compiler_params=pltpu.CompilerParams(
            vmem_limit_bytes=62 * 1024 * 1024,
        ),
    )(emap, tmap, af, h, w_down)


NW = 32
NTP = 32


def _sc_plan_gather_body(eids_hbm, x_hbm, xs_hbm, pos_hbm, emap_hbm, tmap_hbm,
                         af_hbm, eid_v, posA_v, posB_v, plan_v,
                         r0, r1, r2, sr0, sr1, sr2, ss0, ss1, ss2):
    nc = 2
    wid = lax.axis_index("s") * nc + lax.axis_index("c")
    lanes = lax.iota(jnp.int32, 16)
    pltpu.sync_copy(eids_hbm, eid_v)
    zero = jnp.zeros((16,), jnp.int32)
    HC = T // NW

    def _count_w(w, carry):
        cnt_all, baseA, baseB = carry
        snapA = jnp.where(w == wid, cnt_all, zero)
        snapB = jnp.where(w == NW + wid, cnt_all, zero)
        chunk_cnt = zero
        for k in range(HC // 16):
            v = eid_v[pl.ds(w * HC + k * 16, 16)]
            for e in range(E):
                pc = jnp.sum((v == e).astype(jnp.int32))
                chunk_cnt = chunk_cnt + jnp.where(lanes == e, pc, 0)
        return cnt_all + chunk_cnt, baseA + snapA, baseB + snapB

    cnt_all, baseA, baseB = lax.fori_loop(0, 2 * NW, _count_w, (zero, zero, zero))
    ntiles = (cnt_all + TM - 1) // TM
    inc = plsc.cumsum(ntiles)
    tstart = (inc - ntiles) * TM
    tok0 = wid * HC
    for slot, mybase, pos_v in ((0, tstart + baseA, posA_v), (1, tstart + baseB, posB_v)):
        run = zero
        for k in range(HC // 16):
            v = eid_v[pl.ds(slot * T + tok0 + k * 16, 16)]
            pos_k = zero
            for e in range(E):
                m = v == e
                mi = m.astype(jnp.int32)
                pref = plsc.cumsum(mi) - mi
                base_sc = jnp.sum(jnp.where(lanes == e, mybase + run, 0))
                pos_k = jnp.where(m, base_sc + pref, pos_k)
                run = run + jnp.where(lanes == e, jnp.sum(mi), 0)
            pos_v[pl.ds(k * 16, 16)] = pos_k
        pltpu.sync_copy(pos_v, pos_hbm.at[pl.ds(slot * T + tok0, HC)])
    GR = 16
    NG = HC // GR
    rows = (r0, r1, r2)
    sr = (sr0, sr1, sr2)
    ss = (ss0, ss1, ss2)
    reads = {}
    for j in range(3):
        reads[j] = pltpu.async_copy(x_hbm.at[pl.ds(tok0 + j * GR, GR)], rows[j], sr[j])
    pend = {}
    for k in range(NG):
        b = k % 3
        reads.pop(k).wait()
        ia = posA_v[pl.ds(k * GR, GR)]
        ib = posB_v[pl.ds(k * GR, GR)]
        pend[k] = (pltpu.async_copy(rows[b], xs_hbm.at[ia], ss[b]),
                   pltpu.async_copy(rows[b], xs_hbm.at[ib], ss[b]))
        if k + 3 < NG:
            for hnd in pend.pop(k):
                hnd.wait()
            reads[k + 3] = pltpu.async_copy(
                x_hbm.at[pl.ds(tok0 + (k + 3) * GR, GR)], rows[b], sr[b])
    for pair in pend.values():
        for hnd in pair:
            hnd.wait()
    @pl.when(wid == 0)
    def _():
        nact = jnp.sum(jnp.where(lanes == E - 1, inc, 0))
        e_last = zero
        for e in range(E):
            te = jnp.sum(jnp.where(lanes == e, inc, 0))
            e_last = e_last + jnp.where(nact - 1 >= te, 1, 0)
        for half in range(NTP // 16):
            j = lax.iota(jnp.int32, 16) + half * 16
            ej = zero
            for e in range(E):
                te = jnp.sum(jnp.where(lanes == e, inc, 0))
                ej = ej + jnp.where(j >= te, 1, 0)
            act = j < nact
            plan_v[pl.ds(0, 16)] = jnp.where(act, ej, e_last)
            plan_v[pl.ds(16, 16)] = jnp.minimum(j, nact - 1)
            plan_v[pl.ds(32, 16)] = act.astype(jnp.int32)
            pltpu.sync_copy(plan_v.at[pl.ds(0, 16)], emap_hbm.at[pl.ds(half * 16, 16)])
            pltpu.sync_copy(plan_v.at[pl.ds(16, 16)], tmap_hbm.at[pl.ds(half * 16, 16)])
            pltpu.sync_copy(plan_v.at[pl.ds(32, 16)], af_hbm.at[pl.ds(half * 16, 16)])


def _sc_plan_gather(eids, x):
    mesh = plsc.VectorSubcoreMesh(core_axis_name="c", subcore_axis_name="s")
    f = pl.kernel(
        _sc_plan_gather_body,
        mesh=mesh,
        out_type=[
            jax.ShapeDtypeStruct((ROWS, D), jnp.float32),
            jax.ShapeDtypeStruct((2 * T,), jnp.int32),
            jax.ShapeDtypeStruct((NTP,), jnp.int32),
            jax.ShapeDtypeStruct((NTP,), jnp.int32),
            jax.ShapeDtypeStruct((NTP,), jnp.int32),
        ],
        scratch_types=[
            pltpu.VMEM((2 * T,), jnp.int32),
            pltpu.VMEM((T // NW,), jnp.int32),
            pltpu.VMEM((T // NW,), jnp.int32),
            pltpu.VMEM((48,), jnp.int32),
            pltpu.VMEM((16, D), jnp.float32),
            pltpu.VMEM((16, D), jnp.float32),
            pltpu.VMEM((16, D), jnp.float32),
        ] + [pltpu.SemaphoreType.DMA] * 6,
        compiler_params=pltpu.CompilerParams(needs_layout_passes=False),
    )
    return f(eids, x)


def _sc_combine_body(x_hbm, ys_hbm, pos_hbm, w_hbm, y_hbm,
                     posA_v, posB_v, wAB_v,
                     rx0, rx1, ra0, ra1, rb0, rb1,
                     sx0, sx1, sa0, sa1, sb0, sb1, sw0, sw1):
    nc = 2
    wid = lax.axis_index("s") * nc + lax.axis_index("c")
    ntok = T // NW
    GR = 8
    NG = ntok // GR
    tok0 = wid * ntok
    rx = (rx0, rx1)
    ra = (ra0, ra1)
    rb = (rb0, rb1)
    sx = (sx0, sx1)
    sa = (sa0, sa1)
    sb = (sb0, sb1)
    sw = (sw0, sw1)
    pltpu.sync_copy(pos_hbm.at[pl.ds(tok0, ntok)], posA_v)
    pltpu.sync_copy(pos_hbm.at[pl.ds(T + tok0, ntok)], posB_v)
    pltpu.sync_copy(w_hbm.at[pl.ds(tok0, ntok)], wAB_v.at[pl.ds(0, ntok)])
    pltpu.sync_copy(w_hbm.at[pl.ds(T + tok0, ntok)], wAB_v.at[pl.ds(ntok, ntok)])

    def start_inputs(g, b):
        return (
            pltpu.async_copy(x_hbm.at[pl.ds(tok0 + g * GR, GR)], rx[b], sx[b]),
            pltpu.async_copy(ys_hbm.at[posA_v.at[pl.ds(g * GR, GR)]], ra[b], sa[b]),
            pltpu.async_copy(ys_hbm.at[posB_v.at[pl.ds(g * GR, GR)]], rb[b], sb[b]),
        )

    pend_in = {0: start_inputs(0, 0)}
    pend_w = {}
    for g in range(NG):
        b = g % 2
        for hnd in pend_in.pop(g):
            hnd.wait()
        if g + 1 < NG:
            bn = 1 - b
            if bn in pend_w:
                pend_w.pop(bn).wait()
            pend_in[g + 1] = start_inputs(g + 1, bn)
        wa16 = wAB_v[pl.ds((g // 2) * 16, 16)]
        wb16 = wAB_v[pl.ds(ntok + (g // 2) * 16, 16)]
        for r in range(GR):
            lane = r + GR * (g % 2)
            ridx = jnp.full((16,), lane, jnp.int32)
            was = lax.gather(wa16, ridx[:, None],
                             lax.GatherDimensionNumbers((), (0,), (0,)), (1,),
                             mode=lax.GatherScatterMode.PROMISE_IN_BOUNDS)
            wbs = lax.gather(wb16, ridx[:, None],
                             lax.GatherDimensionNumbers((), (0,), (0,)), (1,),
                             mode=lax.GatherScatterMode.PROMISE_IN_BOUNDS)

            def _col(ci, _):
                for u in range(8):
                    sl = pl.ds(ci * 128 + u * 16, 16)
                    rx[b][r, sl] = rx[b][r, sl] + was * ra[b][r, sl] + wbs * rb[b][r, sl]
                return 0

            lax.fori_loop(0, D // 128, _col, 0)
        pend_w[b] = pltpu.async_copy(rx[b], y_hbm.at[pl.ds(tok0 + g * GR, GR)], sw[b])
    for hnd in pend_w.values():
        hnd.wait()


def _sc_combine(x, ys, pos, w01):
    mesh = plsc.VectorSubcoreMesh(core_axis_name="c", subcore_axis_name="s")
    ntok = T // NW
    row = pltpu.VMEM((8, D), jnp.float32)
    f = pl.kernel(
        _sc_combine_body,
        mesh=mesh,
        out_type=jax.ShapeDtypeStruct((T, D), jnp.float32),
        scratch_types=[
            pltpu.VMEM((ntok,), jnp.int32),
            pltpu.VMEM((ntok,), jnp.int32),
            pltpu.VMEM((2 * ntok,), jnp.float32),
            row, row, row, row, row, row,
        ] + [pltpu.SemaphoreType.DMA] * 8,
        compiler_params=pltpu.CompilerParams(needs_layout_passes=False),
    )
    return f(x, ys, pos, w01)


def _plan(idx0, idx1):
    eids = jnp.concatenate([idx0, idx1])
    tokens = jnp.concatenate([jnp.arange(T, dtype=jnp.int32)] * 2)
    counts = jnp.bincount(eids, length=E)
    ntiles = (counts + TM - 1) // TM
    tile_cum = jnp.cumsum(ntiles)
    tstart = (tile_cum - ntiles) * TM
    nact = tile_cum[-1]
    perm = jnp.argsort(eids, stable=True)
    cnt_excl = jnp.cumsum(counts) - counts
    se = eids[perm]
    rank = jnp.arange(2 * T, dtype=jnp.int32) - cnt_excl[se]
    row_sorted = tstart[se].astype(jnp.int32) + rank
    pos = jnp.zeros((2 * T,), jnp.int32).at[perm].set(row_sorted)
    src = jnp.zeros((ROWS,), jnp.int32).at[row_sorted].set(tokens[perm])
    j = jnp.arange(NT, dtype=jnp.int32)
    ej = jnp.searchsorted(tile_cum, j, side="right").astype(jnp.int32)
    af = (j < nact).astype(jnp.int32)
    emap = jnp.where(af == 1, jnp.minimum(ej, E - 1), jnp.minimum(ej, E - 1))
    emap = jnp.where(af == 1, emap, emap[jnp.maximum(nact - 1, 0)])
    tmap = jnp.minimum(j, nact - 1).astype(jnp.int32)
    return pos[:T], pos[T:], src, emap, tmap, af


def kernel(hidden_states, Wg, W_gate, W_up, W_down):
    orig_shape = hidden_states.shape
    x = hidden_states.reshape(-1, orig_shape[-1])
    wg_pad = jnp.zeros((D, 128), jnp.float32).at[:, :E].set(Wg)
    eids, w01 = _router(x, wg_pad)
    xs, pos, emap, tmap, af = _sc_plan_gather(eids, x)
    ys = _grouped_ffn(xs, W_gate, W_up, W_down, emap, tmap, af)
    y = _sc_combine(x, ys, pos, w01)
    return y.reshape(orig_shape)


# speedup vs baseline: 1.0445x; 1.0017x over previous
import jax
import jax.numpy as jnp
from jax import lax
from jax.experimental import pallas as pl
from jax.experimental.pallas import tpu as pltpu
from jax.experimental.pallas import tpu_sc as plsc

E = 8
TOP_K = 2
D = 2048
F = 1408
T = 2048
TM = 256
NT = (T * TOP_K) // TM + (E - 1)
ROWS = NT * TM


def _router_body(x_ref, wg_ref, eids_ref, w01_ref):
    l = jnp.dot(x_ref[...], wg_ref[...], preferred_element_type=jnp.float32)
    lane = jax.lax.broadcasted_iota(jnp.int32, l.shape, 1)
    l = jnp.where(lane < E, l, -1e30)
    m1 = jnp.max(l, axis=1, keepdims=True)
    a1 = jnp.min(jnp.where(l == m1, lane, E), axis=1, keepdims=True)
    l2 = jnp.where(lane == a1, -1e30, l)
    m2 = jnp.max(l2, axis=1, keepdims=True)
    a2 = jnp.min(jnp.where(l2 == m2, lane, E), axis=1, keepdims=True)
    w0 = 1.0 / (1.0 + jnp.exp(m2 - m1))
    eids_ref[0] = a1
    eids_ref[1] = a2
    w01_ref[0] = w0
    w01_ref[1] = 1.0 - w0


def _router(x, wg_pad):
    eids, w01 = pl.pallas_call(
        _router_body,
        grid=(T // TM,),
        in_specs=[
            pl.BlockSpec((TM, D), lambda i: (i, 0)),
            pl.BlockSpec((D, 128), lambda i: (0, 0)),
        ],
        out_specs=[
            pl.BlockSpec((2, TM, 1), lambda i: (0, i, 0)),
            pl.BlockSpec((2, TM, 1), lambda i: (0, i, 0)),
        ],
        out_shape=[
            jax.ShapeDtypeStruct((2, T, 1), jnp.int32),
            jax.ShapeDtypeStruct((2, T, 1), jnp.float32),
        ],
    )(x, wg_pad)
    return eids.reshape(2 * T), w01.reshape(2 * T)


def _gateup_body(emap_ref, tmap_ref, af_ref, xs_ref, wg_ref, wu_ref, h_ref):
    i = pl.program_id(0)

    @pl.when(af_ref[i] == 1)
    def _():
        x = xs_ref[...]
        g = jnp.dot(x, wg_ref[0], preferred_element_type=jnp.float32)
        u = jnp.dot(x, wu_ref[0], preferred_element_type=jnp.float32)
        h_ref[...] = (g * jax.nn.sigmoid(g) * u).astype(jnp.bfloat16)


def _down_body(emap_ref, tmap_ref, af_ref, h_ref, wd_ref, out_ref):
    i = pl.program_id(0)

    @pl.when(af_ref[i] == 1)
    def _():
        h = h_ref[...].astype(jnp.float32)
        out_ref[...] = jnp.dot(h, wd_ref[0], preferred_element_type=jnp.float32)


def _grouped_ffn(xs, w_gate, w_up, w_down, emap, tmap, af):
    h = pl.pallas_call(
        _gateup_body,
        grid_spec=pltpu.PrefetchScalarGridSpec(
            num_scalar_prefetch=3,
            grid=(NT,),
            in_specs=[
                pl.BlockSpec((TM, D), lambda i, em, tm, af_: (tm[i], 0)),
                pl.BlockSpec((1, D, F), lambda i, em, tm, af_: (em[i], 0, 0)),
                pl.BlockSpec((1, D, F), lambda i, em, tm, af_: (em[i], 0, 0)),
            ],
            out_specs=pl.BlockSpec((TM, F), lambda i, em, tm, af_: (tm[i], 0)),
        ),
        out_shape=jax.ShapeDtypeStruct((ROWS, F), jnp.bfloat16),
        compiler_params=pltpu.CompilerParams(
            vmem_limit_bytes=62 * 1024 * 1024,
        ),
    )(emap, tmap, af, xs, w_gate, w_up)
    return pl.pallas_call(
        _down_body,
        grid_spec=pltpu.PrefetchScalarGridSpec(
            num_scalar_prefetch=3,
            grid=(NT,),
            in_specs=[
                pl.BlockSpec((TM, F), lambda i, em, tm, af_: (tm[i], 0)),
                pl.BlockSpec((1, F, D), lambda i, em, tm, af_: (em[i], 0, 0)),
            ],
            out_specs=pl.BlockSpec((TM, D), lambda i, em, tm, af_: (tm[i], 0)),
        ),
        out_shape=jax.ShapeDtypeStruct((ROWS, D), jnp.float32),
        compiler_params=pltpu.CompilerParams(
            vmem_limit_bytes=62 * 1024 * 1024,
        ),
    )(emap, tmap, af, h, w_down)


NW = 32
NTP = 32


def _sc_plan_gather_body(eids_hbm, x_hbm, xs_hbm, pos_hbm, emap_hbm, tmap_hbm,
                         af_hbm, eid_v, posA_v, posB_v, plan_v,
                         r0, r1, r2, sr0, sr1, sr2, ss0, ss1, ss2):
    nc = 2
    wid = lax.axis_index("s") * nc + lax.axis_index("c")
    lanes = lax.iota(jnp.int32, 16)
    pltpu.sync_copy(eids_hbm, eid_v)
    zero = jnp.zeros((16,), jnp.int32)
    HC = T // NW

    def _count_w(w, carry):
        cnt_all, baseA, baseB = carry
        snapA = jnp.where(w == wid, cnt_all, zero)
        snapB = jnp.where(w == NW + wid, cnt_all, zero)
        chunk_cnt = zero
        for k in range(HC // 16):
            v = eid_v[pl.ds(w * HC + k * 16, 16)]
            for e in range(E):
                pc = jnp.sum((v == e).astype(jnp.int32))
                chunk_cnt = chunk_cnt + jnp.where(lanes == e, pc, 0)
        return cnt_all + chunk_cnt, baseA + snapA, baseB + snapB

    cnt_all, baseA, baseB = lax.fori_loop(0, 2 * NW, _count_w, (zero, zero, zero))
    ntiles = (cnt_all + TM - 1) // TM
    inc = plsc.cumsum(ntiles)
    tstart = (inc - ntiles) * TM
    tok0 = wid * HC
    for slot, mybase, pos_v in ((0, tstart + baseA, posA_v), (1, tstart + baseB, posB_v)):
        run = zero
        for k in range(HC // 16):
            v = eid_v[pl.ds(slot * T + tok0 + k * 16, 16)]
            pos_k = zero
            for e in range(E):
                m = v == e
                mi = m.astype(jnp.int32)
                pref = plsc.cumsum(mi) - mi
                base_sc = jnp.sum(jnp.where(lanes == e, mybase + run, 0))
                pos_k = jnp.where(m, base_sc + pref, pos_k)
                run = run + jnp.where(lanes == e, jnp.sum(mi), 0)
            pos_v[pl.ds(k * 16, 16)] = pos_k
        pltpu.sync_copy(pos_v, pos_hbm.at[pl.ds(slot * T + tok0, HC)])
    GR = 16
    NG = HC // GR
    rows = (r0, r1, r2)
    sr = (sr0, sr1, sr2)
    ss = (ss0, ss1, ss2)
    reads = {}
    for j in range(3):
        reads[j] = pltpu.async_copy(x_hbm.at[pl.ds(tok0 + j * GR, GR)], rows[j], sr[j])
    pend = {}
    for k in range(NG):
        b = k % 3
        reads.pop(k).wait()
        ia = posA_v[pl.ds(k * GR, GR)]
        ib = posB_v[pl.ds(k * GR, GR)]
        pend[k] = (pltpu.async_copy(rows[b], xs_hbm.at[ia], ss[b]),
                   pltpu.async_copy(rows[b], xs_hbm.at[ib], ss[b]))
        if k + 3 < NG:
            for hnd in pend.pop(k):
                hnd.wait()
            reads[k + 3] = pltpu.async_copy(
                x_hbm.at[pl.ds(tok0 + (k + 3) * GR, GR)], rows[b], sr[b])
    for pair in pend.values():
        for hnd in pair:
            hnd.wait()
    @pl.when(wid == 0)
    def _():
        nact = jnp.sum(jnp.where(lanes == E - 1, inc, 0))
        e_last = zero
        for e in range(E):
            te = jnp.sum(jnp.where(lanes == e, inc, 0))
            e_last = e_last + jnp.where(nact - 1 >= te, 1, 0)
        for half in range(NTP // 16):
            j = lax.iota(jnp.int32, 16) + half * 16
            ej = zero
            for e in range(E):
                te = jnp.sum(jnp.where(lanes == e, inc, 0))
                ej = ej + jnp.where(j >= te, 1, 0)
            act = j < nact
            plan_v[pl.ds(0, 16)] = jnp.where(act, ej, e_last)
            plan_v[pl.ds(16, 16)] = jnp.minimum(j, nact - 1)
            plan_v[pl.ds(32, 16)] = act.astype(jnp.int32)
            pltpu.sync_copy(plan_v.at[pl.ds(0, 16)], emap_hbm.at[pl.ds(half * 16, 16)])
            pltpu.sync_copy(plan_v.at[pl.ds(16, 16)], tmap_hbm.at[pl.ds(half * 16, 16)])
            pltpu.sync_copy(plan_v.at[pl.ds(32, 16)], af_hbm.at[pl.ds(half * 16, 16)])


def _sc_plan_gather(eids, x):
    mesh = plsc.VectorSubcoreMesh(core_axis_name="c", subcore_axis_name="s")
    f = pl.kernel(
        _sc_plan_gather_body,
        mesh=mesh,
        out_type=[
            jax.ShapeDtypeStruct((ROWS, D), jnp.float32),
            jax.ShapeDtypeStruct((2 * T,), jnp.int32),
            jax.ShapeDtypeStruct((NTP,), jnp.int32),
            jax.ShapeDtypeStruct((NTP,), jnp.int32),
            jax.ShapeDtypeStruct((NTP,), jnp.int32),
        ],
        scratch_types=[
            pltpu.VMEM((2 * T,), jnp.int32),
            pltpu.VMEM((T // NW,), jnp.int32),
            pltpu.VMEM((T // NW,), jnp.int32),
            pltpu.VMEM((48,), jnp.int32),
            pltpu.VMEM((16, D), jnp.float32),
            pltpu.VMEM((16, D), jnp.float32),
            pltpu.VMEM((16, D), jnp.float32),
        ] + [pltpu.SemaphoreType.DMA] * 6,
        compiler_params=pltpu.CompilerParams(needs_layout_passes=False),
    )
    return f(eids, x)


def _sc_combine_body(x_hbm, ys_hbm, pos_hbm, w_hbm, y_hbm,
                     posA_v, posB_v, wAB_v,
                     rx0, rx1, ra0, ra1, rb0, rb1,
                     sx0, sx1, sa0, sa1, sb0, sb1, sw0, sw1):
    nc = 2
    wid = lax.axis_index("s") * nc + lax.axis_index("c")
    ntok = T // NW
    GR = 8
    NG = ntok // GR
    tok0 = wid * ntok
    rx = (rx0, rx1)
    ra = (ra0, ra1)
    rb = (rb0, rb1)
    sx = (sx0, sx1)
    sa = (sa0, sa1)
    sb = (sb0, sb1)
    sw = (sw0, sw1)
    pltpu.sync_copy(pos_hbm.at[pl.ds(tok0, ntok)], posA_v)
    pltpu.sync_copy(pos_hbm.at[pl.ds(T + tok0, ntok)], posB_v)
    pltpu.sync_copy(w_hbm.at[pl.ds(tok0, ntok)], wAB_v.at[pl.ds(0, ntok)])
    pltpu.sync_copy(w_hbm.at[pl.ds(T + tok0, ntok)], wAB_v.at[pl.ds(ntok, ntok)])

    def start_inputs(g, b):
        return (
            pltpu.async_copy(x_hbm.at[pl.ds(tok0 + g * GR, GR)], rx[b], sx[b]),
            pltpu.async_copy(ys_hbm.at[posA_v.at[pl.ds(g * GR, GR)]], ra[b], sa[b]),
            pltpu.async_copy(ys_hbm.at[posB_v.at[pl.ds(g * GR, GR)]], rb[b], sb[b]),
        )

    pend_in = {0: start_inputs(0, 0)}
    pend_w = {}
    for g in range(NG):
        b = g % 2
        for hnd in pend_in.pop(g):
            hnd.wait()
        if g + 1 < NG:
            bn = 1 - b
            if bn in pend_w:
                pend_w.pop(bn).wait()
            pend_in[g + 1] = start_inputs(g + 1, bn)
        wa16 = wAB_v[pl.ds((g // 2) * 16, 16)]
        wb16 = wAB_v[pl.ds(ntok + (g // 2) * 16, 16)]
        for r in range(GR):
            lane = r + GR * (g % 2)
            ridx = jnp.full((16,), lane, jnp.int32)
            was = lax.gather(wa16, ridx[:, None],
                             lax.GatherDimensionNumbers((), (0,), (0,)), (1,),
                             mode=lax.GatherScatterMode.PROMISE_IN_BOUNDS)
            wbs = lax.gather(wb16, ridx[:, None],
                             lax.GatherDimensionNumbers((), (0,), (0,)), (1,),
                             mode=lax.GatherScatterMode.PROMISE_IN_BOUNDS)

            def _col(ci, _):
                for u in range(8):
                    sl = pl.ds(ci * 128 + u * 16, 16)
                    rx[b][r, sl] = rx[b][r, sl] + was * ra[b][r, sl] + wbs * rb[b][r, sl]
                return 0

            lax.fori_loop(0, D // 128, _col, 0)
        pend_w[b] = pltpu.async_copy(rx[b], y_hbm.at[pl.ds(tok0 + g * GR, GR)], sw[b])
    for hnd in pend_w.values():
        hnd.wait()


def _sc_combine(x, ys, pos, w01):
    mesh = plsc.VectorSubcoreMesh(core_axis_name="c", subcore_axis_name="s")
    ntok = T // NW
    row = pltpu.VMEM((8, D), jnp.float32)
    f = pl.kernel(
        _sc_combine_body,
        mesh=mesh,
        out_type=jax.ShapeDtypeStruct((T, D), jnp.float32),
        scratch_types=[
            pltpu.VMEM((ntok,), jnp.int32),
            pltpu.VMEM((ntok,), jnp.int32),
            pltpu.VMEM((2 * ntok,), jnp.float32),
            row, row, row, row, row, row,
        ] + [pltpu.SemaphoreType.DMA] * 8,
        compiler_params=pltpu.CompilerParams(needs_layout_passes=False),
    )
    return f(x, ys, pos, w01)


def kernel(hidden_states, Wg, W_gate, W_up, W_down):
    orig_shape = hidden_states.shape
    x = hidden_states.reshape(-1, orig_shape[-1])
    wg_pad = jnp.zeros((D, 128), jnp.float32).at[:, :E].set(Wg)
    eids, w01 = _router(x, wg_pad)
    xs, pos, emap, tmap, af = _sc_plan_gather(eids, x)
    ys = _grouped_ffn(xs, W_gate, W_up, W_down, emap, tmap, af)
    y = _sc_combine(x, ys, pos, w01)
    return y.reshape(orig_shape)
